# baseline (speedup 1.0000x reference)
import functools

import jax
import jax.numpy as jnp
import numpy as np
from jax import lax
from jax.experimental import pallas as pl
from jax.experimental.pallas import tpu as pltpu
from jax.experimental.pallas import tpu_sc as plsc

N_POINTS = 262144
N_LEVELS = 8
FEAT_DIM = 8
LOG2_T = 19
T = 1 << LOG2_T
BASE_RES = 16
PER_LEVEL_SCALE = 1.5
BOUND = 1.0

_RES = [float(np.floor(BASE_RES * (PER_LEVEL_SCALE ** l))) for l in range(N_LEVELS)]
_K1 = np.int32(np.uint32(2654435761).astype(np.int32))
_K2 = np.int32(np.uint32(805459861).astype(np.int32))

NC = 2
NS = 16
NW = NC * NS
PPW = N_POINTS // NW
B = 512
NCHUNK = PPW // B
NG = B // 16
NIDX = 8 * B
NSTREAM = NIDX // 128

HV_LEN = 272


def _hv_pack(style_scale_w, style_scale_b, style_shift_w, style_shift_b,
             out_w, out_b):
    ow = out_w[:, 0]
    v0 = ow * (1.0 + style_scale_b)
    g = style_scale_w * ow[None, :]
    u = style_shift_w @ ow
    s0 = style_shift_b @ ow + out_b[0]
    res = jnp.asarray(_RES, dtype=jnp.float32)
    return jnp.concatenate([
        v0, g[0], g[1], g[2], u, s0[None], res,
        jnp.zeros((HV_LEN - 268,), jnp.float32),
    ]).astype(jnp.float32)


def _kernel_body(x_hbm, tab_hbm, hv_hbm, out_hbm,
                 hv_v, xr_v, x_v, xn_v, idx_v, w_v, rows_v, acc_v,
                 sem0, sem1):
    wid = lax.axis_index("s") * NC + lax.axis_index("c")
    base = wid * PPW
    pltpu.sync_copy(hv_hbm, hv_v)
    iota = lax.iota(jnp.int32, 16)
    l0 = wid * 0

    def a_phase(l, half):
        res = hv_v[pl.ds(260 + l, 16)][0]
        ibase = half * NIDX

        def grp_idx(j, _):
            xn0 = xn_v[0, pl.ds(j * 16, 16)]
            xn1 = xn_v[1, pl.ds(j * 16, 16)]
            xn2 = xn_v[2, pl.ds(j * 16, 16)]
            p0 = xn0 * res
            p1 = xn1 * res
            p2 = xn2 * res
            i0 = p0.astype(jnp.int32)
            i1 = p1.astype(jnp.int32)
            i2 = p2.astype(jnp.int32)
            f0 = p0 - i0.astype(jnp.float32)
            f1 = p1 - i1.astype(jnp.float32)
            f2 = p2 - i2.astype(jnp.float32)
            fb0 = 1.0 - f0
            fb1 = 1.0 - f1
            fb2 = 1.0 - f2
            for corner in range(8):
                b0 = corner & 1
                b1 = (corner >> 1) & 1
                b2 = (corner >> 2) & 1
                c0 = i0 + b0 if b0 else i0
                c1 = i1 + b1 if b1 else i1
                c2 = i2 + b2 if b2 else i2
                h = c0 ^ (c1 * _K1) ^ (c2 * _K2)
                hidx = (h & jnp.int32(T - 1)) + l * T
                w = ((f0 if b0 else fb0)
                     * (f1 if b1 else fb1)
                     * (f2 if b2 else fb2))
                off = ibase + corner * B + j * 16
                idx_v[pl.ds(off, 16)] = hidx
                w_v[pl.ds(off, 16)] = w
            return 0

        lax.fori_loop(0, NG, grp_idx, 0)

    def fire(l, half, sem):
        ibase = half * NIDX
        for i in range(NSTREAM):
            pltpu.async_copy(
                tab_hbm.at[idx_v.at[pl.ds(ibase + i * 128, 128)]],
                rows_v.at[pl.ds(ibase + i * 128, 128), :],
                sem)

    def drain(l, half, sem):
        ibase = half * NIDX
        for i in range(NSTREAM):
            pltpu.make_async_copy(
                tab_hbm.at[pl.ds(0, 128)],

                rows_v.at[pl.ds(ibase + i * 128, 128), :],
                sem).wait()

    def c_phase(l, half):
        ibase = half * NIDX
        lo = l * 8
        hv0 = hv_v[pl.ds(lo, 16)]
        hg0 = hv_v[pl.ds(64 + lo, 16)]
        hg1 = hv_v[pl.ds(128 + lo, 16)]
        hg2 = hv_v[pl.ds(192 + lo, 16)]

        def grp_acc(j, _):
            x0 = x_v[0, pl.ds(j * 16, 16)]
            x1 = x_v[1, pl.ds(j * 16, 16)]
            x2 = x_v[2, pl.ds(j * 16, 16)]
            out16 = acc_v[pl.ds(j * 16, 16)]
            rb = ibase + j * 16 + iota
            for f in range(8):
                tf = (hv0[f] + hg0[f] * x0
                      + hg1[f] * x1 + hg2[f] * x2)
                colf = jnp.full((16,), f, jnp.int32)
                feat = jnp.zeros((16,), jnp.float32)
                for corner in range(8):
                    rows16 = plsc.load_gather(
                        rows_v, [corner * B + rb, colf])
                    wc = w_v[pl.ds(ibase + corner * B + j * 16, 16)]
                    feat = feat + wc * rows16
                out16 = out16 + feat * tf
            acc_v[pl.ds(j * 16, 16)] = out16
            return 0

        lax.fori_loop(0, NG, grp_acc, 0)

    def chunk_body(ck, _):
        cbase = base + ck * B
        pltpu.sync_copy(x_hbm.at[pl.ds(cbase, B), :], xr_v)
        hvu = hv_v[pl.ds(256, 16)]

        def norm_body(j, _):
            rows = j * 16 + iota
            for k in range(3):
                v = plsc.load_gather(
                    xr_v, [rows, jnp.full((16,), k, jnp.int32)])
                x_v[k, pl.ds(j * 16, 16)] = v
                vn = jnp.minimum(
                    jnp.maximum((v + BOUND) * (0.5 / BOUND), 0.0), 1.0 - 1e-6)
                xn_v[k, pl.ds(j * 16, 16)] = vn
            x0 = x_v[0, pl.ds(j * 16, 16)]
            x1 = x_v[1, pl.ds(j * 16, 16)]
            x2 = x_v[2, pl.ds(j * 16, 16)]
            a = x0 * hvu[0] + x1 * hvu[1] + x2 * hvu[2] + hvu[3]
            acc_v[pl.ds(j * 16, 16)] = a
            return 0

        lax.fori_loop(0, NG, norm_body, 0)

        a_phase(l0, 0)
        fire(l0, 0, sem0)

        def pair_body(lp, _):
            le = 2 * lp
            lo_ = le + 1
            a_phase(lo_, 1)
            fire(lo_, 1, sem1)
            drain(le, 0, sem0)
            c_phase(le, 0)

            @pl.when(lp < (N_LEVELS // 2 - 1))
            def _():
                a_phase(le + 2, 0)
                fire(le + 2, 0, sem0)

            drain(lo_, 1, sem1)
            c_phase(lo_, 1)
            return 0

        lax.fori_loop(0, N_LEVELS // 2, pair_body, 0)

        def fin_body(j, _):
            acc_v[pl.ds(j * 16, 16)] = (
                acc_v[pl.ds(j * 16, 16)] * (1.0 / N_LEVELS))
            return 0

        lax.fori_loop(0, NG, fin_body, 0)
        pltpu.sync_copy(acc_v, out_hbm.at[pl.ds(cbase, B)])
        return 0

    lax.fori_loop(0, NCHUNK, chunk_body, 0)


NTAB = N_LEVELS * T


def _relayout_body(tf_hbm, out_hbm):
    wid = lax.axis_index("s") * NC + lax.axis_index("c")
    ch = (NTAB * FEAT_DIM) // NW
    off = wid * ch
    pltpu.sync_copy(tf_hbm.at[pl.ds(off, ch)], out_hbm.at[pl.ds(off, ch)])


@jax.jit
def _relayout(tab_flat):
    mesh = plsc.VectorSubcoreMesh(core_axis_name="c", subcore_axis_name="s")
    k = functools.partial(
        pl.kernel, mesh=mesh,
        out_type=jax.ShapeDtypeStruct((NTAB * FEAT_DIM,), jnp.float32),
        compiler_params=pltpu.CompilerParams(
            needs_layout_passes=False, use_tc_tiling_on_sc=False),
    )(_relayout_body)
    return k(tab_flat).reshape(NTAB, FEAT_DIM)


@jax.jit
def _run(x, tab3, hv):
    mesh = plsc.VectorSubcoreMesh(core_axis_name="c", subcore_axis_name="s")
    k = functools.partial(
        pl.kernel, mesh=mesh,
        out_type=jax.ShapeDtypeStruct((N_POINTS,), jnp.float32),
        scratch_types=[
            pltpu.VMEM((HV_LEN,), jnp.float32),
            pltpu.VMEM((B, 3), jnp.float32),
            pltpu.VMEM((3, B), jnp.float32),
            pltpu.VMEM((3, B), jnp.float32),
            pltpu.VMEM((2 * NIDX,), jnp.int32),
            pltpu.VMEM((2 * NIDX,), jnp.float32),
            pltpu.VMEM((2 * NIDX, FEAT_DIM), jnp.float32),
            pltpu.VMEM((B,), jnp.float32),
            pltpu.SemaphoreType.DMA,
            pltpu.SemaphoreType.DMA,
        ],
        compiler_params=pltpu.CompilerParams(
            needs_layout_passes=False, use_tc_tiling_on_sc=False,
            skip_device_barrier=True),
    )(_kernel_body)
    return k(x, tab3, hv)


def kernel(x, tables, style_scale_w, style_scale_b, style_shift_w,
           style_shift_b, out_w, out_b):
    hv = _hv_pack(style_scale_w, style_scale_b, style_shift_w,
                  style_shift_b, out_w, out_b)
    tab2d = _relayout(tables.reshape(-1))
    out = _run(x, tab2d, hv)
    return out[:, None]


# speedup vs baseline: 2.3523x; 2.3523x over previous
import functools

import jax
import jax.numpy as jnp
import numpy as np
from jax import lax
from jax.experimental import pallas as pl
from jax.experimental.pallas import tpu as pltpu
from jax.experimental.pallas import tpu_sc as plsc

N_POINTS = 262144
N_LEVELS = 8
FEAT_DIM = 8
LOG2_T = 19
T = 1 << LOG2_T
BASE_RES = 16
PER_LEVEL_SCALE = 1.5
BOUND = 1.0

_RES = [float(np.floor(BASE_RES * (PER_LEVEL_SCALE ** l))) for l in range(N_LEVELS)]
_K1 = np.int32(np.uint32(2654435761).astype(np.int32))
_K2 = np.int32(np.uint32(805459861).astype(np.int32))

NC = 2
NS = 16
NW = NC * NS
PPW = N_POINTS // NW
B = 512
NCHUNK = PPW // B
NG = B // 16
NIDX = 8 * B
NSTREAM = NIDX // 128

HV_LEN = 272


def _hv_pack(style_scale_w, style_scale_b, style_shift_w, style_shift_b,
             out_w, out_b):
    ow = out_w[:, 0]
    v0 = ow * (1.0 + style_scale_b)
    g = style_scale_w * ow[None, :]
    u = style_shift_w @ ow
    s0 = style_shift_b @ ow + out_b[0]
    res = jnp.asarray(_RES, dtype=jnp.float32)
    return jnp.concatenate([
        v0, g[0], g[1], g[2], u, s0[None], res,
        jnp.zeros((HV_LEN - 268,), jnp.float32),
    ]).astype(jnp.float32)


def _kernel_body(x_hbm, tab_hbm, hv_hbm, out_hbm,
                 hv_v, xr_v, x_v, xn_v, idx_v, w_v, rows_v, acc_v,
                 sem0, sem1):
    wid = lax.axis_index("s") * NC + lax.axis_index("c")
    base = wid * PPW
    pltpu.sync_copy(hv_hbm, hv_v)
    iota = lax.iota(jnp.int32, 16)
    l0 = wid * 0

    def a_phase(l, half):
        res = hv_v[pl.ds(260 + l, 16)][0]
        ibase = half * NIDX

        def grp_idx(j, _):
            xn0 = xn_v[0, pl.ds(j * 16, 16)]
            xn1 = xn_v[1, pl.ds(j * 16, 16)]
            xn2 = xn_v[2, pl.ds(j * 16, 16)]
            p0 = xn0 * res
            p1 = xn1 * res
            p2 = xn2 * res
            i0 = p0.astype(jnp.int32)
            i1 = p1.astype(jnp.int32)
            i2 = p2.astype(jnp.int32)
            f0 = p0 - i0.astype(jnp.float32)
            f1 = p1 - i1.astype(jnp.float32)
            f2 = p2 - i2.astype(jnp.float32)
            fb0 = 1.0 - f0
            fb1 = 1.0 - f1
            fb2 = 1.0 - f2
            for corner in range(8):
                b0 = corner & 1
                b1 = (corner >> 1) & 1
                b2 = (corner >> 2) & 1
                c0 = i0 + b0 if b0 else i0
                c1 = i1 + b1 if b1 else i1
                c2 = i2 + b2 if b2 else i2
                h = c0 ^ (c1 * _K1) ^ (c2 * _K2)
                hidx = h & jnp.int32(T - 1)
                w = ((f0 if b0 else fb0)
                     * (f1 if b1 else fb1)
                     * (f2 if b2 else fb2))
                off = ibase + corner * B + j * 16
                idx_v[pl.ds(off, 16)] = hidx
                w_v[pl.ds(off, 16)] = w
            return 0

        lax.fori_loop(0, NG, grp_idx, 0)

    def fire(l, half, sem):
        ibase = half * NIDX
        for i in range(NSTREAM):
            pltpu.async_copy(
                tab_hbm.at[l].at[idx_v.at[pl.ds(ibase + i * 128, 128)]],
                rows_v.at[pl.ds(ibase + i * 128, 128), :],
                sem)

    def drain(l, half, sem):
        ibase = half * NIDX
        for i in range(NSTREAM):
            pltpu.make_async_copy(
                tab_hbm.at[l].at[pl.ds(0, 128)],

                rows_v.at[pl.ds(ibase + i * 128, 128), :],
                sem).wait()

    def c_phase(l, half):
        ibase = half * NIDX
        lo = l * 8
        hv0 = hv_v[pl.ds(lo, 16)]
        hg0 = hv_v[pl.ds(64 + lo, 16)]
        hg1 = hv_v[pl.ds(128 + lo, 16)]
        hg2 = hv_v[pl.ds(192 + lo, 16)]

        def grp_acc(j, _):
            x0 = x_v[0, pl.ds(j * 16, 16)]
            x1 = x_v[1, pl.ds(j * 16, 16)]
            x2 = x_v[2, pl.ds(j * 16, 16)]
            out16 = acc_v[pl.ds(j * 16, 16)]
            rb = ibase + j * 16 + iota
            for f in range(8):
                tf = (hv0[f] + hg0[f] * x0
                      + hg1[f] * x1 + hg2[f] * x2)
                colf = jnp.full((16,), f, jnp.int32)
                feat = jnp.zeros((16,), jnp.float32)
                for corner in range(8):
                    rows16 = plsc.load_gather(
                        rows_v, [corner * B + rb, colf])
                    wc = w_v[pl.ds(ibase + corner * B + j * 16, 16)]
                    feat = feat + wc * rows16
                out16 = out16 + feat * tf
            acc_v[pl.ds(j * 16, 16)] = out16
            return 0

        lax.fori_loop(0, NG, grp_acc, 0)

    def chunk_body(ck, _):
        cbase = base + ck * B
        pltpu.sync_copy(x_hbm.at[pl.ds(cbase, B), :], xr_v)
        hvu = hv_v[pl.ds(256, 16)]

        def norm_body(j, _):
            rows = j * 16 + iota
            for k in range(3):
                v = plsc.load_gather(
                    xr_v, [rows, jnp.full((16,), k, jnp.int32)])
                x_v[k, pl.ds(j * 16, 16)] = v
                vn = jnp.minimum(
                    jnp.maximum((v + BOUND) * (0.5 / BOUND), 0.0), 1.0 - 1e-6)
                xn_v[k, pl.ds(j * 16, 16)] = vn
            x0 = x_v[0, pl.ds(j * 16, 16)]
            x1 = x_v[1, pl.ds(j * 16, 16)]
            x2 = x_v[2, pl.ds(j * 16, 16)]
            a = x0 * hvu[0] + x1 * hvu[1] + x2 * hvu[2] + hvu[3]
            acc_v[pl.ds(j * 16, 16)] = a
            return 0

        lax.fori_loop(0, NG, norm_body, 0)

        a_phase(l0, 0)
        fire(l0, 0, sem0)

        def pair_body(lp, _):
            le = 2 * lp
            lo_ = le + 1
            a_phase(lo_, 1)
            fire(lo_, 1, sem1)
            drain(le, 0, sem0)
            c_phase(le, 0)

            @pl.when(lp < (N_LEVELS // 2 - 1))
            def _():
                a_phase(le + 2, 0)
                fire(le + 2, 0, sem0)

            drain(lo_, 1, sem1)
            c_phase(lo_, 1)
            return 0

        lax.fori_loop(0, N_LEVELS // 2, pair_body, 0)

        def fin_body(j, _):
            acc_v[pl.ds(j * 16, 16)] = (
                acc_v[pl.ds(j * 16, 16)] * (1.0 / N_LEVELS))
            return 0

        lax.fori_loop(0, NG, fin_body, 0)
        pltpu.sync_copy(acc_v, out_hbm.at[pl.ds(cbase, B)])
        return 0

    lax.fori_loop(0, NCHUNK, chunk_body, 0)


@jax.jit
def _run(x, tab3, hv):
    mesh = plsc.VectorSubcoreMesh(core_axis_name="c", subcore_axis_name="s")
    k = functools.partial(
        pl.kernel, mesh=mesh,
        out_type=jax.ShapeDtypeStruct((N_POINTS,), jnp.float32),
        scratch_types=[
            pltpu.VMEM((HV_LEN,), jnp.float32),
            pltpu.VMEM((B, 3), jnp.float32),
            pltpu.VMEM((3, B), jnp.float32),
            pltpu.VMEM((3, B), jnp.float32),
            pltpu.VMEM((2 * NIDX,), jnp.int32),
            pltpu.VMEM((2 * NIDX,), jnp.float32),
            pltpu.VMEM((2 * NIDX, FEAT_DIM), jnp.float32),
            pltpu.VMEM((B,), jnp.float32),
            pltpu.SemaphoreType.DMA,
            pltpu.SemaphoreType.DMA,
        ],
        compiler_params=pltpu.CompilerParams(
            needs_layout_passes=False, use_tc_tiling_on_sc=False,
            skip_device_barrier=True),
    )(_kernel_body)
    return k(x, tab3, hv)


def kernel(x, tables, style_scale_w, style_scale_b, style_shift_w,
           style_shift_b, out_w, out_b):
    hv = _hv_pack(style_scale_w, style_scale_b, style_shift_w,
                  style_shift_b, out_w, out_b)
    out = _run(x, tables, hv)
    return out[:, None]
